# Initial kernel scaffold; baseline (speedup 1.0000x reference)
import jax
import jax.numpy as jnp
from jax.experimental import pallas as pl


def kernel(x_nchw, conv_w_oihw, gamma, beta):
    raise NotImplementedError("write your pallas kernel here")


# speedup vs baseline: 1.0013x; 1.0013x over previous
import functools

import jax
import jax.numpy as jnp
from jax import lax
from jax.experimental import pallas as pl
from jax.experimental.pallas import tpu as pltpu


def _conv_acc(x_ref, w_ref, slab_ref, *, nb, c_out, W, HW, K):
    col = lax.broadcasted_iota(jnp.int32, (1, 1, HW), 2) % W
    mask_l = (col != 0).astype(jnp.float32)
    mask_r = (col != (W - 1)).astype(jnp.float32)

    for kh in range(3):
        for kw in range(3):
            start = kh * W + kw
            v = x_ref[:, :, start:start + HW]
            if kw == 0:
                v = v * mask_l
            elif kw == 2:
                v = v * mask_r
            t = kh * 3 + kw
            slab_ref[:, 8 * t:8 * t + 3, :] = v

    w_b = jnp.broadcast_to(w_ref[...][None], (nb, c_out, K))
    return lax.dot_general(
        w_b, slab_ref[...],
        dimension_numbers=(((2,), (1,)), ((0,), (0,))),
        preferred_element_type=jnp.float32)


def _stats_kernel(x_ref, w_ref, sum_ref, sq_ref, slab_ref,
                  *, nb, c_out, W, HW, K):
    j = pl.program_id(1)

    @pl.when(j == 0)
    def _init():
        slab_ref[...] = jnp.zeros_like(slab_ref)
        sum_ref[...] = jnp.zeros_like(sum_ref)
        sq_ref[...] = jnp.zeros_like(sq_ref)

    acc = _conv_acc(x_ref, w_ref, slab_ref, nb=nb, c_out=c_out, W=W, HW=HW,
                    K=K)
    psum = jnp.sum(jnp.sum(acc, axis=2, keepdims=True), axis=0)
    psq = jnp.sum(jnp.sum(acc * acc, axis=2, keepdims=True), axis=0)
    sum_ref[...] += psum[None]
    sq_ref[...] += psq[None]


def _finalize_kernel(sum_ref, sq_ref, w_ref, g_ref, b_ref,
                     ws_ref, shift_ref, *, m_total, eps):
    inv_m = 1.0 / float(m_total)
    s = sum_ref[0] + sum_ref[1]
    q = sq_ref[0] + sq_ref[1]
    mean = s * inv_m
    var = jnp.maximum(q * inv_m - mean * mean, 0.0)
    inv_std = lax.rsqrt(var + eps)
    scale = g_ref[...] * inv_std
    ws_ref[...] = w_ref[...] * scale
    shift_ref[...] = b_ref[...] - mean * scale


def _apply_kernel(x_ref, w_ref, shift_ref, o_ref, slab_ref,
                  *, nb, c_out, W, HW, K):
    @pl.when(pl.program_id(1) == 0)
    def _init():
        slab_ref[...] = jnp.zeros_like(slab_ref)

    acc = _conv_acc(x_ref, w_ref, slab_ref, nb=nb, c_out=c_out, W=W, HW=HW,
                    K=K)
    o_ref[...] = acc + shift_ref[...][None]


def _round_up(v, m):
    return (v + m - 1) // m * m


def kernel(x_nchw, conv_w_oihw, gamma, beta):
    eps = 1e-5
    N, C_in, H, W = x_nchw.shape
    C_out = conv_w_oihw.shape[0]
    HW = H * W
    K = 72
    Lx = (H + 2) * W + 2
    Lx_pad = _round_up(Lx, 128)
    vmem_limit = ((64 << 20) * 3) // 4

    nb = 32
    while N % (2 * nb) != 0:
        nb //= 2
    steps_half = N // (2 * nb)

    xf = x_nchw.astype(jnp.float32)
    x_hp = jnp.pad(xf, ((0, 0), (0, 0), (1, 1), (0, 0)))
    x_flat = x_hp.reshape(N, C_in, (H + 2) * W)
    x_in = jnp.pad(x_flat, ((0, 0), (0, 0), (1, 1 + Lx_pad - Lx)))

    w_t = jnp.transpose(conv_w_oihw.astype(jnp.float32), (0, 2, 3, 1))
    w_t = w_t.reshape(C_out, 9, C_in)
    w72 = jnp.pad(w_t, ((0, 0), (0, 0), (0, 8 - C_in))).reshape(C_out, K)
    g2 = gamma.reshape(C_out, 1).astype(jnp.float32)
    b2 = beta.reshape(C_out, 1).astype(jnp.float32)

    stats = functools.partial(_stats_kernel, nb=nb, c_out=C_out, W=W, HW=HW,
                              K=K)
    sums, sqs = pl.pallas_call(
        stats,
        out_shape=(jax.ShapeDtypeStruct((2, C_out, 1), jnp.float32),
                   jax.ShapeDtypeStruct((2, C_out, 1), jnp.float32)),
        grid=(2, steps_half),
        in_specs=[
            pl.BlockSpec((nb, C_in, Lx_pad),
                         lambda c, j, sh=steps_half: (c * sh + j, 0, 0)),
            pl.BlockSpec((C_out, K), lambda c, j: (0, 0)),
        ],
        out_specs=(pl.BlockSpec((1, C_out, 1), lambda c, j: (c, 0, 0)),
                   pl.BlockSpec((1, C_out, 1), lambda c, j: (c, 0, 0))),
        scratch_shapes=[pltpu.VMEM((nb, K, HW), jnp.float32)],
        compiler_params=pltpu.CompilerParams(
            dimension_semantics=("parallel", "arbitrary"),
            vmem_limit_bytes=vmem_limit),
    )(x_in, w72)

    fin = functools.partial(_finalize_kernel, m_total=N * H * W, eps=eps)
    w72s, shift = pl.pallas_call(
        fin,
        out_shape=(jax.ShapeDtypeStruct((C_out, K), jnp.float32),
                   jax.ShapeDtypeStruct((C_out, 1), jnp.float32)),
    )(sums, sqs, w72, g2, b2)

    apply_k = functools.partial(_apply_kernel, nb=nb, c_out=C_out, W=W, HW=HW,
                                K=K)
    out_flat = pl.pallas_call(
        apply_k,
        out_shape=jax.ShapeDtypeStruct((N, C_out, HW), jnp.float32),
        grid=(2, steps_half),
        in_specs=[
            pl.BlockSpec((nb, C_in, Lx_pad),
                         lambda c, j, sh=steps_half: (c * sh + j, 0, 0)),
            pl.BlockSpec((C_out, K), lambda c, j: (0, 0)),
            pl.BlockSpec((C_out, 1), lambda c, j: (0, 0)),
        ],
        out_specs=pl.BlockSpec(
            (nb, C_out, HW),
            lambda c, j, sh=steps_half: (c * sh + j, 0, 0)),
        scratch_shapes=[pltpu.VMEM((nb, K, HW), jnp.float32)],
        compiler_params=pltpu.CompilerParams(
            dimension_semantics=("parallel", "arbitrary"),
            vmem_limit_bytes=vmem_limit),
    )(x_in, w72s, shift)

    return out_flat.reshape(N, C_out, H, W)


# speedup vs baseline: 1.3634x; 1.3616x over previous
import functools

import jax
import jax.numpy as jnp
from jax import lax
from jax.experimental import pallas as pl
from jax.experimental.pallas import tpu as pltpu


def _conv_stats_kernel(x_ref, w_ref, y_ref, sum_ref, sq_ref, slab_ref,
                       *, nb, c_out, W, HW, K):
    j = pl.program_id(0)

    @pl.when(j == 0)
    def _init():
        slab_ref[...] = jnp.zeros_like(slab_ref)
        sum_ref[...] = jnp.zeros_like(sum_ref)
        sq_ref[...] = jnp.zeros_like(sq_ref)

    col = lax.broadcasted_iota(jnp.int32, (1, 1, HW), 2) % W
    mask_l = (col != 0).astype(jnp.float32)
    mask_r = (col != (W - 1)).astype(jnp.float32)

    for kh in range(3):
        for kw in range(3):
            start = kh * W + kw
            v = x_ref[:, :, start:start + HW]
            if kw == 0:
                v = v * mask_l
            elif kw == 2:
                v = v * mask_r
            t = kh * 3 + kw
            slab_ref[:, 8 * t:8 * t + 3, :] = v

    w_b = jnp.broadcast_to(w_ref[...][None], (nb, c_out, K))
    acc = lax.dot_general(
        w_b, slab_ref[...],
        dimension_numbers=(((2,), (1,)), ((0,), (0,))),
        preferred_element_type=jnp.float32)

    sum_ref[...] += jnp.sum(jnp.sum(acc, axis=2, keepdims=True), axis=0)
    sq_ref[...] += jnp.sum(jnp.sum(acc * acc, axis=2, keepdims=True), axis=0)
    y_ref[...] = acc.astype(jnp.bfloat16)


def _affine_kernel(y_ref, sum_ref, sq_ref, g_ref, b_ref, o_ref,
                   *, m_total, eps):
    inv_m = 1.0 / float(m_total)
    mean = sum_ref[...] * inv_m
    var = jnp.maximum(sq_ref[...] * inv_m - mean * mean, 0.0)
    inv_std = lax.rsqrt(var + eps)
    scale = g_ref[...] * inv_std
    shift = b_ref[...] - mean * scale
    y = y_ref[...].astype(jnp.float32)
    o_ref[...] = y * scale[None] + shift[None]


def _round_up(v, m):
    return (v + m - 1) // m * m


def kernel(x_nchw, conv_w_oihw, gamma, beta):
    eps = 1e-5
    N, C_in, H, W = x_nchw.shape
    C_out = conv_w_oihw.shape[0]
    HW = H * W
    K = 72
    Lx = (H + 2) * W + 2
    Lx_pad = _round_up(Lx, 128)
    vmem_limit = ((64 << 20) * 3) // 4

    nb = 64
    while N % nb != 0:
        nb //= 2
    steps = N // nb
    nb2 = 128
    while N % nb2 != 0:
        nb2 //= 2
    steps2 = N // nb2

    xf = x_nchw.astype(jnp.float32)
    x_hp = jnp.pad(xf, ((0, 0), (0, 0), (1, 1), (0, 0)))
    x_flat = x_hp.reshape(N, C_in, (H + 2) * W)
    x_in = jnp.pad(x_flat, ((0, 0), (0, 0), (1, 1 + Lx_pad - Lx)))

    w_t = jnp.transpose(conv_w_oihw.astype(jnp.float32), (0, 2, 3, 1))
    w_t = w_t.reshape(C_out, 9, C_in)
    w72 = jnp.pad(w_t, ((0, 0), (0, 0), (0, 8 - C_in))).reshape(C_out, K)
    g2 = gamma.reshape(C_out, 1).astype(jnp.float32)
    b2 = beta.reshape(C_out, 1).astype(jnp.float32)

    conv_stats = functools.partial(_conv_stats_kernel, nb=nb, c_out=C_out,
                                   W=W, HW=HW, K=K)
    y16, sums, sqs = pl.pallas_call(
        conv_stats,
        out_shape=(jax.ShapeDtypeStruct((N, C_out, HW), jnp.bfloat16),
                   jax.ShapeDtypeStruct((C_out, 1), jnp.float32),
                   jax.ShapeDtypeStruct((C_out, 1), jnp.float32)),
        grid=(steps,),
        in_specs=[
            pl.BlockSpec((nb, C_in, Lx_pad), lambda j: (j, 0, 0)),
            pl.BlockSpec((C_out, K), lambda j: (0, 0)),
        ],
        out_specs=(pl.BlockSpec((nb, C_out, HW), lambda j: (j, 0, 0)),
                   pl.BlockSpec((C_out, 1), lambda j: (0, 0)),
                   pl.BlockSpec((C_out, 1), lambda j: (0, 0))),
        scratch_shapes=[pltpu.VMEM((nb, K, HW), jnp.float32)],
        compiler_params=pltpu.CompilerParams(
            dimension_semantics=("arbitrary",),
            vmem_limit_bytes=vmem_limit),
    )(x_in, w72)

    aff = functools.partial(_affine_kernel, m_total=N * H * W, eps=eps)
    out_flat = pl.pallas_call(
        aff,
        out_shape=jax.ShapeDtypeStruct((N, C_out, HW), jnp.float32),
        grid=(steps2,),
        in_specs=[
            pl.BlockSpec((nb2, C_out, HW), lambda j: (j, 0, 0)),
            pl.BlockSpec((C_out, 1), lambda j: (0, 0)),
            pl.BlockSpec((C_out, 1), lambda j: (0, 0)),
            pl.BlockSpec((C_out, 1), lambda j: (0, 0)),
            pl.BlockSpec((C_out, 1), lambda j: (0, 0)),
        ],
        out_specs=pl.BlockSpec((nb2, C_out, HW), lambda j: (j, 0, 0)),
        compiler_params=pltpu.CompilerParams(
            dimension_semantics=("arbitrary",),
            vmem_limit_bytes=vmem_limit),
    )(y16, sums, sqs, g2, b2)

    return out_flat.reshape(N, C_out, H, W)
